# baseline (speedup 1.0000x reference)
import jax
import jax.numpy as jnp
from jax import lax
from jax.experimental import pallas as pl
from jax.experimental.pallas import tpu as pltpu

D_IN = 700
LATENT = 64
K = 1024
BATCH = 8192
BB = 1024


def _leaky(v):
    return jnp.where(v > 0, v, 0.2 * v)


def _vqvae_body(x_ref, We0_ref, be0_ref, We1_ref, be1_ref, We2_ref, be2_ref,
                Wd0_ref, bd0_ref, Wd1_ref, bd1_ref, Wd2_ref, bd2_ref, cb_ref,
                out_ref):
    x = x_ref[...]
    h = _leaky(jnp.dot(x, We0_ref[...], preferred_element_type=jnp.float32)
               + be0_ref[...])
    h = _leaky(jnp.dot(h, We1_ref[...], preferred_element_type=jnp.float32)
               + be1_ref[...])
    z = (jnp.dot(h, We2_ref[...], preferred_element_type=jnp.float32)
         + be2_ref[...])

    cb = cb_ref[...]
    sim = jnp.dot(z, cb, preferred_element_type=jnp.float32)
    zsq = jnp.sum(z * z, axis=1, keepdims=True)
    csq = jnp.sum(cb * cb, axis=0, keepdims=True)
    dist = (zsq + csq) - 2.0 * sim

    m = jnp.min(dist, axis=1, keepdims=True)
    iota = lax.broadcasted_iota(jnp.int32, dist.shape, 1)
    idx = jnp.min(jnp.where(dist == m, iota, K), axis=1, keepdims=True)
    onehot = (iota == idx).astype(jnp.float32)
    q = lax.dot_general(onehot, cb, (((1,), (1,)), ((), ())),
                        preferred_element_type=jnp.float32,
                        precision=lax.Precision.HIGHEST)
    q = z + (q - z)

    h = _leaky(jnp.dot(q, Wd0_ref[...], preferred_element_type=jnp.float32)
               + bd0_ref[...])
    h = _leaky(jnp.dot(h, Wd1_ref[...], preferred_element_type=jnp.float32)
               + bd1_ref[...])
    out_ref[...] = (jnp.dot(h, Wd2_ref[...], preferred_element_type=jnp.float32)
                    + bd2_ref[...])


def kernel(x, We0, be0, We1, be1, We2, be2, Wd0, bd0, Wd1, bd1, Wd2, bd2,
           codebook):
    full = lambda a: pl.BlockSpec(a.shape, lambda i: (0,) * a.ndim)
    b2 = lambda b: b.reshape(1, -1)
    grid = BATCH // BB
    return pl.pallas_call(
        _vqvae_body,
        grid=(grid,),
        in_specs=[
            pl.BlockSpec((BB, D_IN), lambda i: (i, 0)),
            full(We0), full(b2(be0)), full(We1), full(b2(be1)),
            full(We2), full(b2(be2)),
            full(Wd0), full(b2(bd0)), full(Wd1), full(b2(bd1)),
            full(Wd2), full(b2(bd2)),
            full(codebook),
        ],
        out_specs=pl.BlockSpec((BB, D_IN), lambda i: (i, 0)),
        out_shape=jax.ShapeDtypeStruct((BATCH, D_IN), jnp.float32),
        compiler_params=pltpu.CompilerParams(
            dimension_semantics=("parallel",),
        ),
    )(x, We0, b2(be0), We1, b2(be1), We2, b2(be2),
      Wd0, b2(bd0), Wd1, b2(bd1), Wd2, b2(bd2), codebook)


# speedup vs baseline: 1.4748x; 1.4748x over previous
import jax
import jax.numpy as jnp
from jax import lax
from jax.experimental import pallas as pl
from jax.experimental.pallas import tpu as pltpu

D_IN = 700
LATENT = 64
K = 1024
BATCH = 8192
BB = 1024


def _leaky(v):
    return jnp.where(v > 0, v, 0.2 * v)


def _vqvae_body(xt_ref, We0_ref, be0_ref, We1_ref, be1_ref, We2t_ref, be2_ref,
                Wd0_ref, bd0_ref, Wd1_ref, bd1_ref, Wd2t_ref, bd2_ref, cb_ref,
                out_ref):
    x = xt_ref[...].T
    h = _leaky(jnp.dot(x, We0_ref[...], preferred_element_type=jnp.float32)
               + be0_ref[...])
    h = _leaky(jnp.dot(h, We1_ref[...], preferred_element_type=jnp.float32)
               + be1_ref[...])
    z = (lax.dot_general(h, We2t_ref[...], (((1,), (1,)), ((), ())),
                         preferred_element_type=jnp.float32)
         + be2_ref[...])

    cb = cb_ref[...]
    sim = jnp.dot(z, cb, preferred_element_type=jnp.float32)
    zsq = jnp.sum(z * z, axis=1, keepdims=True)
    csq = jnp.sum(cb * cb, axis=0, keepdims=True)
    dist = (zsq + csq) - 2.0 * sim

    m = jnp.min(dist, axis=1, keepdims=True)
    iota = lax.broadcasted_iota(jnp.int32, dist.shape, 1)
    idx = jnp.min(jnp.where(dist == m, iota, K), axis=1, keepdims=True)
    onehot = (iota == idx).astype(jnp.float32)
    q = lax.dot_general(onehot, cb, (((1,), (1,)), ((), ())),
                        preferred_element_type=jnp.float32,
                        precision=lax.Precision.HIGHEST)
    q = z + (q - z)

    h = _leaky(jnp.dot(q, Wd0_ref[...], preferred_element_type=jnp.float32)
               + bd0_ref[...])
    h = _leaky(jnp.dot(h, Wd1_ref[...], preferred_element_type=jnp.float32)
               + bd1_ref[...])
    out = (lax.dot_general(h, Wd2t_ref[...], (((1,), (1,)), ((), ())),
                           preferred_element_type=jnp.float32)
           + bd2_ref[...])
    out_ref[...] = out.T


def kernel(x, We0, be0, We1, be1, We2, be2, Wd0, bd0, Wd1, bd1, Wd2, bd2,
           codebook):
    full = lambda a: pl.BlockSpec(a.shape, lambda i: (0,) * a.ndim)
    grid = BATCH // BB
    outt = pl.pallas_call(
        _vqvae_body,
        grid=(grid,),
        in_specs=[
            pl.BlockSpec((D_IN, BB), lambda i: (0, i)),
            full(We0), full(be0), full(We1), full(be1),
            full(We2.T), full(be2),
            full(Wd0), full(bd0), full(Wd1), full(bd1),
            full(Wd2.T), full(bd2),
            full(codebook),
        ],
        out_specs=pl.BlockSpec((D_IN, BB), lambda i: (0, i)),
        out_shape=jax.ShapeDtypeStruct((D_IN, BATCH), jnp.float32),
        compiler_params=pltpu.CompilerParams(
            dimension_semantics=("arbitrary",),
        ),
    )(x.T, We0, be0, We1, be1, We2.T, be2,
      Wd0, bd0, Wd1, bd1, Wd2.T, bd2, codebook)
    return outt.T


# speedup vs baseline: 1.6884x; 1.1448x over previous
import jax
import jax.numpy as jnp
from jax import lax
from jax.experimental import pallas as pl
from jax.experimental.pallas import tpu as pltpu

D_IN = 700
LATENT = 64
K = 1024
BATCH = 8192
BB = 1024


def _leaky(v):
    return jnp.where(v > 0, v, 0.2 * v)


def _vqvae_body(xt_ref, We0_ref, be0_ref, We1_ref, be1_ref, We2t_ref, be2_ref,
                Wd0_ref, bd0_ref, Wd1_ref, bd1_ref, Wd2t_ref, bd2_ref, cb_ref,
                out_ref, q_ref):
    x = xt_ref[...].T
    h = _leaky(jnp.dot(x, We0_ref[...], preferred_element_type=jnp.float32)
               + be0_ref[...])
    h = _leaky(jnp.dot(h, We1_ref[...], preferred_element_type=jnp.float32)
               + be1_ref[...])
    z = (lax.dot_general(h, We2t_ref[...], (((1,), (1,)), ((), ())),
                         preferred_element_type=jnp.float32)
         + be2_ref[...])

    cb = cb_ref[...]
    sim = jnp.dot(z, cb, preferred_element_type=jnp.float32)
    zsq = jnp.sum(z * z, axis=1, keepdims=True)
    csq = jnp.sum(cb * cb, axis=0, keepdims=True)
    dist = (zsq + csq) - 2.0 * sim

    m = jnp.min(dist, axis=1, keepdims=True)
    mask = (dist == m).astype(jnp.float32)
    rowcnt = jnp.dot(mask, jnp.ones((K, 8), jnp.float32),
                     preferred_element_type=jnp.float32)
    q_ref[...] = lax.dot_general(mask, cb, (((1,), (1,)), ((), ())),
                                 preferred_element_type=jnp.float32,
                                 precision=lax.Precision.HIGHEST)

    @pl.when(jnp.max(rowcnt) != 1.0)
    def _ties():
        iota = lax.broadcasted_iota(jnp.int32, dist.shape, 1)
        idx = jnp.min(jnp.where(dist == m, iota, K), axis=1, keepdims=True)
        onehot = (iota == idx).astype(jnp.float32)
        q_ref[...] = lax.dot_general(onehot, cb, (((1,), (1,)), ((), ())),
                                     preferred_element_type=jnp.float32,
                                     precision=lax.Precision.HIGHEST)

    q = z + (q_ref[...] - z)

    h = _leaky(jnp.dot(q, Wd0_ref[...], preferred_element_type=jnp.float32)
               + bd0_ref[...])
    h = _leaky(jnp.dot(h, Wd1_ref[...], preferred_element_type=jnp.float32)
               + bd1_ref[...])
    out = (lax.dot_general(h, Wd2t_ref[...], (((1,), (1,)), ((), ())),
                           preferred_element_type=jnp.float32)
           + bd2_ref[...])
    out_ref[...] = out.T


def kernel(x, We0, be0, We1, be1, We2, be2, Wd0, bd0, Wd1, bd1, Wd2, bd2,
           codebook):
    full = lambda a: pl.BlockSpec(a.shape, lambda i: (0,) * a.ndim)
    grid = BATCH // BB
    outt = pl.pallas_call(
        _vqvae_body,
        grid=(grid,),
        in_specs=[
            pl.BlockSpec((D_IN, BB), lambda i: (0, i)),
            full(We0), full(be0), full(We1), full(be1),
            full(We2.T), full(be2),
            full(Wd0), full(bd0), full(Wd1), full(bd1),
            full(Wd2.T), full(bd2),
            full(codebook),
        ],
        out_specs=pl.BlockSpec((D_IN, BB), lambda i: (0, i)),
        out_shape=jax.ShapeDtypeStruct((D_IN, BATCH), jnp.float32),
        scratch_shapes=[pltpu.VMEM((BB, LATENT), jnp.float32)],
        compiler_params=pltpu.CompilerParams(
            dimension_semantics=("arbitrary",),
        ),
    )(x.T, We0, be0, We1, be1, We2.T, be2,
      Wd0, bd0, Wd1, bd1, Wd2.T, bd2, codebook)
    return outt.T


# speedup vs baseline: 1.6951x; 1.0040x over previous
import jax
import jax.numpy as jnp
from jax import lax
from jax.experimental import pallas as pl
from jax.experimental.pallas import tpu as pltpu

D_IN = 700
LATENT = 64
K = 1024
BATCH = 8192
BB = 1024


def _leaky(v):
    return jnp.where(v > 0, v, 0.2 * v)


def _vqvae_body(xt_ref, We0_ref, be0_ref, We1_ref, be1_ref, We2t_ref, be2_ref,
                Wd0_ref, bd0_ref, Wd1_ref, bd1_ref, Wd2t_ref, bd2_ref, cb_ref,
                out_ref, q_ref):
    x = xt_ref[...].T
    h = _leaky(jnp.dot(x, We0_ref[...], preferred_element_type=jnp.float32)
               + be0_ref[...])
    h = _leaky(jnp.dot(h, We1_ref[...], preferred_element_type=jnp.float32)
               + be1_ref[...])
    z = (lax.dot_general(h, We2t_ref[...], (((1,), (1,)), ((), ())),
                         preferred_element_type=jnp.float32)
         + be2_ref[...])

    cb = cb_ref[...]
    simn = jnp.dot(z, -2.0 * cb, preferred_element_type=jnp.float32)
    zsq = jnp.sum(z * z, axis=1, keepdims=True)
    csq = jnp.sum(cb * cb, axis=0, keepdims=True)
    dist = (zsq + csq) + simn

    m = jnp.min(dist, axis=1, keepdims=True)
    mask = (dist == m).astype(jnp.float32)
    rowcnt = jnp.dot(mask, jnp.ones((K, 8), jnp.float32),
                     preferred_element_type=jnp.float32)
    q_ref[...] = lax.dot_general(mask, cb, (((1,), (1,)), ((), ())),
                                 preferred_element_type=jnp.float32,
                                 precision=lax.Precision.HIGHEST)

    @pl.when(jnp.max(rowcnt) != 1.0)
    def _ties():
        iota = lax.broadcasted_iota(jnp.int32, dist.shape, 1)
        idx = jnp.min(jnp.where(dist == m, iota, K), axis=1, keepdims=True)
        onehot = (iota == idx).astype(jnp.float32)
        q_ref[...] = lax.dot_general(onehot, cb, (((1,), (1,)), ((), ())),
                                     preferred_element_type=jnp.float32,
                                     precision=lax.Precision.HIGHEST)

    q = z + (q_ref[...] - z)

    h = _leaky(jnp.dot(q, Wd0_ref[...], preferred_element_type=jnp.float32)
               + bd0_ref[...])
    h = _leaky(jnp.dot(h, Wd1_ref[...], preferred_element_type=jnp.float32)
               + bd1_ref[...])
    out = (lax.dot_general(h, Wd2t_ref[...], (((1,), (1,)), ((), ())),
                           preferred_element_type=jnp.float32)
           + bd2_ref[...])
    out_ref[...] = out.T


def kernel(x, We0, be0, We1, be1, We2, be2, Wd0, bd0, Wd1, bd1, Wd2, bd2,
           codebook):
    full = lambda a: pl.BlockSpec(a.shape, lambda i: (0,) * a.ndim)
    grid = BATCH // BB
    outt = pl.pallas_call(
        _vqvae_body,
        grid=(grid,),
        in_specs=[
            pl.BlockSpec((D_IN, BB), lambda i: (0, i)),
            full(We0), full(be0), full(We1), full(be1),
            full(We2.T), full(be2),
            full(Wd0), full(bd0), full(Wd1), full(bd1),
            full(Wd2.T), full(bd2),
            full(codebook),
        ],
        out_specs=pl.BlockSpec((D_IN, BB), lambda i: (0, i)),
        out_shape=jax.ShapeDtypeStruct((D_IN, BATCH), jnp.float32),
        scratch_shapes=[pltpu.VMEM((BB, LATENT), jnp.float32)],
        compiler_params=pltpu.CompilerParams(
            dimension_semantics=("arbitrary",),
        ),
    )(x.T, We0, be0, We1, be1, We2.T, be2,
      Wd0, bd0, Wd1, bd1, Wd2.T, bd2, codebook)
    return outt.T
